# baseline (speedup 1.0000x reference)
import functools

import jax
import jax.numpy as jnp
from jax import lax
from jax.experimental import pallas as pl
from jax.experimental.pallas import tpu as pltpu
from jax.experimental.pallas import tpu_sc as plsc

B = 4096
D = 64
F = 128
WPAD = D + F + 16


def kernel(user_idx, movie_idx, movie_feats, user_table, movie_table, fc_w, fc_b):
    info = plsc.get_sparse_core_info()
    nw = info.num_cores * info.num_subcores
    bpw = B // nw
    mesh = plsc.VectorSubcoreMesh(core_axis_name="c", subcore_axis_name="s")

    w_all = jnp.zeros((1, WPAD), jnp.float32)
    w_all = w_all.at[0, : D + F].set(fc_w[0])
    w_all = w_all.at[0, D + F].set(fc_b[0])

    @functools.partial(
        pl.kernel,
        out_type=jax.ShapeDtypeStruct((B,), jnp.float32),
        mesh=mesh,
        scratch_types=[
            pltpu.VMEM((bpw,), jnp.int32),
            pltpu.VMEM((bpw,), jnp.int32),
            pltpu.VMEM((bpw, D), jnp.float32),
            pltpu.VMEM((bpw, D), jnp.float32),
            pltpu.VMEM((bpw, F), jnp.float32),
            pltpu.VMEM((1, WPAD), jnp.float32),
            pltpu.VMEM((bpw,), jnp.float32),
            pltpu.SemaphoreType.DMA,
            pltpu.SemaphoreType.DMA,
        ],
        compiler_params=pltpu.CompilerParams(
            needs_layout_passes=False, use_tc_tiling_on_sc=False),
    )
    def run(uidx_h, midx_h, feats_h, utab_h, mtab_h, w_h, out_h,
            uidx_v, midx_v, u_v, m_v, f_v, w_v, o_v, sem_u, sem_m):
        wid = lax.axis_index("s") * info.num_cores + lax.axis_index("c")
        base = wid * bpw
        pltpu.sync_copy(uidx_h.at[pl.ds(base, bpw)], uidx_v)
        pltpu.sync_copy(midx_h.at[pl.ds(base, bpw)], midx_v)
        cu = pltpu.async_copy(utab_h.at[uidx_v], u_v, sem_u)
        cm = pltpu.async_copy(mtab_h.at[midx_v], m_v, sem_m)
        pltpu.sync_copy(feats_h.at[pl.ds(base, bpw)], f_v)
        pltpu.sync_copy(w_h, w_v)
        cu.wait()
        cm.wait()

        lanes = lax.iota(jnp.int32, 16)
        bias_vec = w_v[0, pl.ds(D + F, 16)]

        def group(g, carry):
            res = jnp.zeros((16,), jnp.float32)
            for r in range(16):
                i = g * 16 + r
                acc = bias_vec
                for c in range(D // 16):
                    acc += (u_v[i, pl.ds(c * 16, 16)]
                            * m_v[i, pl.ds(c * 16, 16)]
                            * w_v[0, pl.ds(c * 16, 16)])
                for c in range(F // 16):
                    acc += (f_v[i, pl.ds(c * 16, 16)]
                            * w_v[0, pl.ds(D + c * 16, 16)])
                res = jnp.where(lanes == r, jnp.sum(acc), res)
            o_v[pl.ds(g * 16, 16)] = res
            return carry

        lax.fori_loop(0, bpw // 16, group, 0)
        pltpu.sync_copy(o_v, out_h.at[pl.ds(base, bpw)])

    return run(user_idx, movie_idx, movie_feats, user_table, movie_table,
               w_all)


# speedup vs baseline: 1.4337x; 1.4337x over previous
import functools

import jax
import jax.numpy as jnp
from jax import lax
from jax.experimental import pallas as pl
from jax.experimental.pallas import tpu as pltpu
from jax.experimental.pallas import tpu_sc as plsc

B = 4096
D = 64
F = 128
WPAD = 256


def kernel(user_idx, movie_idx, movie_feats, user_table, movie_table, fc_w, fc_b):
    info = plsc.get_sparse_core_info()
    nw = info.num_cores * info.num_subcores
    bpw = B // nw
    mesh = plsc.VectorSubcoreMesh(core_axis_name="c", subcore_axis_name="s")

    w_all = jnp.zeros((WPAD,), jnp.float32)
    w_all = w_all.at[: D + F].set(fc_w[0])
    w_all = w_all.at[D + F].set(fc_b[0])

    @functools.partial(
        pl.kernel,
        out_type=jax.ShapeDtypeStruct((B,), jnp.float32),
        mesh=mesh,
        scratch_types=[
            pltpu.VMEM((bpw,), jnp.int32),
            pltpu.VMEM((bpw,), jnp.int32),
            pltpu.VMEM((bpw, D), jnp.float32),
            pltpu.VMEM((bpw, D), jnp.float32),
            pltpu.VMEM((bpw, F), jnp.float32),
            pltpu.VMEM((WPAD,), jnp.float32),
            pltpu.VMEM((bpw,), jnp.float32),
            pltpu.SemaphoreType.DMA,
            pltpu.SemaphoreType.DMA,
        ],
        compiler_params=pltpu.CompilerParams(needs_layout_passes=False),
    )
    def run(uidx_h, midx_h, feats_h, utab_h, mtab_h, w_h, out_h,
            uidx_v, midx_v, u_v, m_v, f_v, w_v, o_v, sem_u, sem_m):
        wid = lax.axis_index("s") * info.num_cores + lax.axis_index("c")
        base = wid * bpw
        pltpu.sync_copy(uidx_h.at[pl.ds(base, bpw)], uidx_v)
        pltpu.sync_copy(midx_h.at[pl.ds(base, bpw)], midx_v)

        def fire(g, carry):
            uv = uidx_v[pl.ds(g * 16, 16)]
            mv = midx_v[pl.ds(g * 16, 16)]
            for r in range(16):
                iu = uv[r]
                im = mv[r]
                pltpu.make_async_copy(
                    utab_h.at[pl.ds(iu, 1), :],
                    u_v.at[pl.ds(g * 16 + r, 1), :], sem_u
                ).start()
                pltpu.make_async_copy(
                    mtab_h.at[pl.ds(im, 1), :],
                    m_v.at[pl.ds(g * 16 + r, 1), :], sem_m
                ).start()
            return carry

        lax.fori_loop(0, bpw // 16, fire, 0)

        pltpu.sync_copy(feats_h.at[pl.ds(base, bpw)], f_v)
        pltpu.sync_copy(w_h, w_v)

        pltpu.make_async_copy(utab_h.at[pl.ds(0, bpw), :], u_v, sem_u).wait()
        pltpu.make_async_copy(mtab_h.at[pl.ds(0, bpw), :], m_v, sem_m).wait()

        lanes = lax.iota(jnp.int32, 16)
        bias_vec = w_v[pl.ds(D + F, 16)]

        def group(g, carry):
            res = jnp.zeros((16,), jnp.float32)
            for r in range(16):
                i = g * 16 + r
                acc = bias_vec
                for c in range(D // 16):
                    acc += (u_v[i, pl.ds(c * 16, 16)]
                            * m_v[i, pl.ds(c * 16, 16)]
                            * w_v[pl.ds(c * 16, 16)])
                for c in range(F // 16):
                    acc += (f_v[i, pl.ds(c * 16, 16)]
                            * w_v[pl.ds(D + c * 16, 16)])
                res = jnp.where(lanes == r, jnp.sum(acc), res)
            o_v[pl.ds(g * 16, 16)] = res
            return carry

        lax.fori_loop(0, bpw // 16, group, 0)
        pltpu.sync_copy(o_v, out_h.at[pl.ds(base, bpw)])

    return run(user_idx, movie_idx, movie_feats, user_table, movie_table,
               w_all)
